# baseline (speedup 1.0000x reference)
import jax
import jax.numpy as jnp
from jax import lax
from jax.experimental import pallas as pl
from jax.experimental.pallas import tpu as pltpu

_CNF = 0.5
_IOU = 0.5
_N = 5000
_NS = 8192
_RS = _NS // 128
_NR = 40
_NPAD = _NR * 128


def _roll(x, shift, lane):
    ax = 1 if lane else 0
    n = x.shape[ax]
    s = shift % n
    if s == 0:
        return x
    if lane:
        return jnp.concatenate([x[:, s:], x[:, :s]], axis=1)
    return jnp.concatenate([x[s:, :], x[:s, :]], axis=0)


def _bitonic_sort(s, idx, payload):
    rows = s.shape[0]
    flat = (lax.broadcasted_iota(jnp.int32, (rows, 128), 0) * 128
            + lax.broadcasted_iota(jnp.int32, (rows, 128), 1))
    n = rows * 128
    k = 2
    while k <= n:
        j = k // 2
        while j >= 1:
            lane = j < 128
            shift = j if lane else j // 128
            bit = (flat & j) != 0
            fwd = (flat & k) == 0

            def partner(a):
                return jnp.where(bit, _roll(a, -shift, lane),
                                 _roll(a, shift, lane))

            ps, pidx = partner(s), partner(idx)
            pb = (ps > s) | ((ps == s) & (pidx < idx))
            take = ~(bit ^ pb ^ fwd)
            s = jnp.where(take, ps, s)
            idx = jnp.where(take, pidx, idx)
            payload = [jnp.where(take, partner(a), a) for a in payload]
            j //= 2
        k *= 2
    return s, idx, payload


def _body(cx_ref, cy_ref, w_ref, h_ref, s_ref, out_ref,
          x1w, y1w, x2w, y2w, aw, sw, kcol_ref, kwide_ref):
    flat = (lax.broadcasted_iota(jnp.int32, (_RS, 128), 0) * 128
            + lax.broadcasted_iota(jnp.int32, (_RS, 128), 1))

    s, _, (cx, cy, w, h) = _bitonic_sort(
        s_ref[...], flat, [cx_ref[...], cy_ref[...], w_ref[...], h_ref[...]])

    s = s[:_NR]
    cx, cy, w, h = cx[:_NR], cy[:_NR], w[:_NR], h[:_NR]
    x1 = cx - w / 2.0
    y1 = cy - h / 2.0
    x2 = cx + w / 2.0
    y2 = cy + h / 2.0
    area = (x2 - x1) * (y2 - y1)

    for r in range(_NR):
        x1w[r] = x1[r:r + 1]
        y1w[r] = y1[r:r + 1]
        x2w[r] = x2[r:r + 1]
        y2w[r] = y2[r:r + 1]
        aw[r] = area[r:r + 1]
        sw[r] = s[r:r + 1]

    kwide_ref[...] = jnp.zeros((_NR, 1, 128), jnp.float32)

    count = jnp.sum((s > _CNF).astype(jnp.int32))
    nb = (count + 127) // 128

    eye = (lax.broadcasted_iota(jnp.int32, (128, 128), 0)
           == lax.broadcasted_iota(jnp.int32, (128, 128), 1)
           ).astype(jnp.float32)
    tri_t = (lax.broadcasted_iota(jnp.int32, (128, 128), 1)
             < lax.broadcasted_iota(jnp.int32, (128, 128), 0)
             ).astype(jnp.float32)

    def tr_row(row):
        return lax.dot_general(eye, row, (((1,), (1,)), ((), ())),
                               preferred_element_type=jnp.float32)

    def outer(j, _):
        jx1, jy1 = tr_row(x1w[j]), tr_row(y1w[j])
        jx2, jy2 = tr_row(x2w[j]), tr_row(y2w[j])
        ja = tr_row(aw[j])
        valid = (tr_row(sw[j]) > _CNF).astype(jnp.float32)

        def overlap(i):
            ix = jnp.maximum(
                0.0, jnp.minimum(jx2, x2w[i]) - jnp.maximum(jx1, x1w[i]))
            iy = jnp.maximum(
                0.0, jnp.minimum(jy2, y2w[i]) - jnp.maximum(jy1, y1w[i]))
            inter = ix * iy
            union = ja + aw[i] - inter
            return jnp.where(inter > _IOU * (union + 1e-9), 1.0, 0.0)

        def cross(i, sup):
            return sup + lax.dot_general(
                overlap(i), kcol_ref[i], (((1,), (0,)), ((), ())),
                preferred_element_type=jnp.float32)

        sup = lax.fori_loop(0, j, cross, jnp.zeros((128, 1), jnp.float32))
        m0 = jnp.where(sup > 0.0, 0.0, valid)

        o_jj = overlap(j) * tri_t

        def cond(c):
            return c[1]

        def bodyf(c):
            kk, _ = c
            sp = lax.dot_general(o_jj, kk, (((1,), (0,)), ((), ())),
                                 preferred_element_type=jnp.float32)
            k2 = jnp.where(sp > 0.0, 0.0, m0)
            return k2, jnp.any(k2 != kk)

        k_fix, _ = lax.while_loop(cond, bodyf, (m0, True))
        kcol_ref[j] = k_fix
        kwide_ref[j] = lax.dot_general(
            k_fix, eye, (((0,), (0,)), ((), ())),
            preferred_element_type=jnp.float32)
        return 0

    lax.fori_loop(0, nb, outer, 0)

    kp = kwide_ref[...]
    out_ref[0] = x1w[...] * kp
    out_ref[1] = y1w[...] * kp
    out_ref[2] = x2w[...] * kp
    out_ref[3] = y2w[...] * kp
    out_ref[4] = sw[...] * kp


@jax.jit
def kernel(boxes, scores):
    pad = _NS - _N
    z = jnp.zeros((pad,), jnp.float32)

    def prep(v):
        return jnp.concatenate([v, z]).reshape(_RS, 128)

    cx = prep(boxes[:, 0])
    cy = prep(boxes[:, 1])
    w = prep(boxes[:, 2])
    h = prep(boxes[:, 3])
    s = jnp.concatenate([scores, jnp.full((pad,), -1.0, jnp.float32)]
                        ).reshape(_RS, 128)

    ow = pl.pallas_call(
        _body,
        out_shape=jax.ShapeDtypeStruct((5, _NR, 1, 128), jnp.float32),
        scratch_shapes=[pltpu.VMEM((_NR, 1, 128), jnp.float32)] * 6
        + [pltpu.VMEM((_NR, 128, 1), jnp.float32),
           pltpu.VMEM((_NR, 1, 128), jnp.float32)],
    )(cx, cy, w, h, s)

    return ow.reshape(5, _NPAD)[:, :_N].T
